# baseline (speedup 1.0000x reference)
import functools

import jax
import jax.numpy as jnp
from jax import lax
from jax.experimental import pallas as pl
from jax.experimental.pallas import tpu as pltpu
from jax.experimental.pallas import tpu_sc as plsc

_NC = 2
_NS = 16
_NW = _NC * _NS
_L = 16

_B = 16384
_D = 32
_BPW = _B // _NW
_CH = 128
_NCH = _BPW // _CH
_NBLK = _BPW // _L


def _rsqrt16(x):
    i = plsc.bitcast(x, jnp.int32)
    y = plsc.bitcast(jnp.int32(0x5F3759DF) - (i >> 1), jnp.float32)
    for _ in range(3):
        y = y * (1.5 - 0.5 * x * y * y)
    return y


def _body(lbl_hbm, hash_hbm, crx_hbm, preset_hbm, out_hbm,
          hidx, pidx, rows, crx, outv, sem):
    c = lax.axis_index("c")
    s = lax.axis_index("s")
    wid = s * _NC + c
    base = wid * _BPW

    pltpu.sync_copy(hash_hbm.at[wid], hidx)

    cps = [pltpu.async_copy(lbl_hbm.at[hidx.at[k]], pidx.at[k], sem)
           for k in range(_NCH)]
    for cp in cps:
        cp.wait()

    cps = [pltpu.async_copy(preset_hbm.at[pidx.at[k]],
                            rows.at[pl.ds(k * _CH, _CH)], sem)
           for k in range(_NCH)]
    pltpu.sync_copy(crx_hbm.at[pl.ds(base, _BPW)], crx)
    for cp in cps:
        cp.wait()

    def blk(b, carry):
        rid = b * _L + lax.iota(jnp.int32, _L)
        acc_d = jnp.zeros((_L,), jnp.float32)
        acc_s = jnp.zeros((_L,), jnp.float32)
        for j in range(_D):
            cj = jnp.full((_L,), j, jnp.int32)
            a = plsc.load_gather(crx, [rid, cj])
            p = plsc.load_gather(rows, [rid, cj])
            acc_s = acc_s + a * a
            acc_d = acc_d + a * p
        outv[pl.ds(b * _L, _L)] = acc_d * _rsqrt16(acc_s)
        return carry

    lax.fori_loop(0, _NBLK, blk, 0)

    pltpu.sync_copy(outv, out_hbm.at[pl.ds(base, _BPW)])


_clf = functools.partial(
    pl.kernel,
    out_type=jax.ShapeDtypeStruct((_B,), jnp.float32),
    mesh=plsc.VectorSubcoreMesh(core_axis_name="c", subcore_axis_name="s"),
    compiler_params=pltpu.CompilerParams(
        needs_layout_passes=False, use_tc_tiling_on_sc=False),
    scratch_types=[
        pltpu.VMEM((_NCH, _CH), jnp.int32),
        pltpu.VMEM((_NCH, _CH), jnp.int32),
        pltpu.VMEM((_BPW, _D), jnp.float32),
        pltpu.VMEM((_BPW, _D), jnp.float32),
        pltpu.VMEM((_BPW,), jnp.float32),
        pltpu.SemaphoreType.DMA,
    ],
)(_body)


def kernel(lbl_indx, hash_map, crx_vect, preset):
    lbl = lbl_indx.astype(jnp.int32)
    hm = hash_map.astype(jnp.int32).reshape(_NW, _NCH, _CH)
    return _clf(lbl, hm, crx_vect, preset)


# speedup vs baseline: 1.4889x; 1.4889x over previous
import functools

import jax
import jax.numpy as jnp
from jax import lax
from jax.experimental import pallas as pl
from jax.experimental.pallas import tpu as pltpu
from jax.experimental.pallas import tpu_sc as plsc

_NC = 2
_NS = 16
_NW = _NC * _NS
_L = 16

_B = 16384
_D = 32
_V = 1000000
_BPW = _B // _NW
_CH = 128
_NCH = _BPW // _CH
_PH = 4
_RPP = _BPW // _PH
_NBLK = _RPP // _L


def _rsqrt16(x):
    i = plsc.bitcast(x, jnp.int32)
    y = plsc.bitcast(jnp.int32(0x5F3759DF) - (i >> 1), jnp.float32)
    for _ in range(3):
        y = y * (1.5 - 0.5 * x * y * y)
    return y


def _body(lbl_hbm, hash_hbm, crxt_hbm, pt_hbm, out_hbm,
          hidx, pidx, g, crxv, outv, sem, gsem):
    c = lax.axis_index("c")
    s = lax.axis_index("s")
    wid = s * _NC + c
    base = wid * _BPW

    pltpu.sync_copy(hash_hbm.at[pl.ds(base, _BPW)], hidx)

    cps = [pltpu.async_copy(lbl_hbm.at[hidx.at[pl.ds(k * _CH, _CH)]],
                            pidx.at[pl.ds(k * _CH, _CH)], sem)
           for k in range(_NCH)]
    for cp in cps:
        cp.wait()

    ccp = pltpu.async_copy(crxt_hbm.at[:, pl.ds(base, _BPW)], crxv, sem)

    iota = lax.iota(jnp.int32, _L)
    ccp.wait()

    def phase(ph, pcarry):
        pbase = ph * _RPP

        def fetch(b, carry):
            vec = pidx[pl.ds(pbase + b * _L, _L)]
            al128 = (vec >> 7) * 128
            sub = (vec >> 4) & 7
            for t in range(_L):
                win = pt_hbm.at[:, :, pl.ds(pl.multiple_of(al128[t], 128), 128)]
                dst = g.at[:, :, pl.ds((b * _L + t) * _L, _L)]

                def mk(s, win=win, dst=dst):
                    def br():
                        pltpu.async_copy(
                            win.at[:, :, pl.ds(s * _L, _L)], dst, gsem)
                    return br

                lax.switch(sub[t], [mk(s) for s in range(8)])
            pltpu.make_async_copy(
                pt_hbm.at[:, :, pl.ds(0, _L * _L)],
                g.at[:, :, pl.ds(b * _L * _L, _L * _L)], gsem).wait()
            return carry

        lax.fori_loop(0, _NBLK, fetch, 0)

        def blk(b, carry):
            o = pbase + b * _L
            low = pidx[pl.ds(o, _L)] & (_L - 1)
            lane = b * (_L * _L) + iota * _L + low
            acc_d = jnp.zeros((_L,), jnp.float32)
            acc_s = jnp.zeros((_L,), jnp.float32)
            for j in range(_D):
                a = crxv[j, pl.ds(o, _L)]
                p = plsc.load_gather(
                    g, [jnp.full((_L,), j // 8, jnp.int32),
                        jnp.full((_L,), j % 8, jnp.int32), lane])
                acc_s = acc_s + a * a
                acc_d = acc_d + a * p
            outv[pl.ds(o, _L)] = acc_d * _rsqrt16(acc_s)
            return carry

        lax.fori_loop(0, _NBLK, blk, 0)
        return pcarry

    lax.fori_loop(0, _PH, phase, 0)

    pltpu.sync_copy(outv, out_hbm.at[pl.ds(base, _BPW)])


_clf = functools.partial(
    pl.kernel,
    out_type=jax.ShapeDtypeStruct((_B,), jnp.float32),
    mesh=plsc.VectorSubcoreMesh(core_axis_name="c", subcore_axis_name="s"),
    compiler_params=pltpu.CompilerParams(
        needs_layout_passes=False, use_tc_tiling_on_sc=True),
    scratch_types=[
        pltpu.VMEM((_BPW,), jnp.int32),
        pltpu.VMEM((_BPW,), jnp.int32),
        pltpu.VMEM((4, 8, _RPP * _L), jnp.float32),
        pltpu.VMEM((_D, _BPW), jnp.float32),
        pltpu.VMEM((_BPW,), jnp.float32),
        pltpu.SemaphoreType.DMA,
        pltpu.SemaphoreType.DMA,
    ],
)(_body)


def kernel(lbl_indx, hash_map, crx_vect, preset):
    lbl = lbl_indx.astype(jnp.int32)
    hm = hash_map.astype(jnp.int32)
    crxt = crx_vect.T
    pt = preset.T.reshape(4, 8, _V)
    return _clf(lbl, hm, crxt, pt)


# speedup vs baseline: 1.5064x; 1.0118x over previous
import functools

import jax
import jax.numpy as jnp
from jax import lax
from jax.experimental import pallas as pl
from jax.experimental.pallas import tpu as pltpu
from jax.experimental.pallas import tpu_sc as plsc

_NC = 2
_NS = 16
_NW = _NC * _NS
_L = 16

_B = 16384
_D = 32
_V = 1000000
_BPW = _B // _NW
_CH = 128
_NCH = _BPW // _CH
_PH = 4
_RPP = _BPW // _PH
_NBLK = _RPP // _L


def _rsqrt16(x):
    i = plsc.bitcast(x, jnp.int32)
    y = plsc.bitcast(jnp.int32(0x5F3759DF) - (i >> 1), jnp.float32)
    for _ in range(3):
        y = y * (1.5 - 0.5 * x * y * y)
    return y


def _body(lbl_hbm, hash_hbm, crxt_hbm, pt_hbm, out_hbm,
          hidx, pidx, g, crxv, outv, sem, gsem):
    c = lax.axis_index("c")
    s = lax.axis_index("s")
    wid = s * _NC + c
    base = wid * _BPW

    pltpu.sync_copy(hash_hbm.at[pl.ds(base, _BPW)], hidx)

    cps = [pltpu.async_copy(lbl_hbm.at[hidx.at[pl.ds(k * _CH, _CH)]],
                            pidx.at[pl.ds(k * _CH, _CH)], sem)
           for k in range(_NCH)]
    for cp in cps:
        cp.wait()

    ccp = pltpu.async_copy(crxt_hbm.at[:, pl.ds(base, _BPW)], crxv, sem)

    iota = lax.iota(jnp.int32, _L)
    ccp.wait()

    def phase(ph, pcarry):
        pbase = ph * _RPP

        def fetch(b, carry):
            vec = pidx[pl.ds(pbase + b * _L, _L)]
            al128 = (vec >> 7) * 128
            sub = (vec >> 4) & 7
            for t in range(_L):
                win = pt_hbm.at[:, :, pl.ds(pl.multiple_of(al128[t], 128), 128)]
                dst = g.at[:, :, pl.ds((b * _L + t) * _L, _L)]

                def mk(s, win=win, dst=dst):
                    def br():
                        pltpu.async_copy(
                            win.at[:, :, pl.ds(s * _L, _L)], dst, gsem)
                    return br

                lax.switch(sub[t], [mk(s) for s in range(8)])

            @pl.when(b >= 2)
            def _():
                pltpu.make_async_copy(
                    pt_hbm.at[:, :, pl.ds(0, _L * _L)],
                    g.at[:, :, pl.ds((b - 2) * _L * _L, _L * _L)],
                    gsem).wait()

            return carry

        lax.fori_loop(0, _NBLK, fetch, 0)
        for tail in (_NBLK - 2, _NBLK - 1):
            pltpu.make_async_copy(
                pt_hbm.at[:, :, pl.ds(0, _L * _L)],
                g.at[:, :, pl.ds(tail * _L * _L, _L * _L)], gsem).wait()

        def blk(b, carry):
            o = pbase + b * _L
            low = pidx[pl.ds(o, _L)] & (_L - 1)
            lane = b * (_L * _L) + iota * _L + low
            acc_d = jnp.zeros((_L,), jnp.float32)
            acc_s = jnp.zeros((_L,), jnp.float32)
            for j in range(_D):
                a = crxv[j, pl.ds(o, _L)]
                p = plsc.load_gather(
                    g, [jnp.full((_L,), j // 8, jnp.int32),
                        jnp.full((_L,), j % 8, jnp.int32), lane])
                acc_s = acc_s + a * a
                acc_d = acc_d + a * p
            outv[pl.ds(o, _L)] = acc_d * _rsqrt16(acc_s)
            return carry

        lax.fori_loop(0, _NBLK, blk, 0)
        return pcarry

    lax.fori_loop(0, _PH, phase, 0)

    pltpu.sync_copy(outv, out_hbm.at[pl.ds(base, _BPW)])


_clf = functools.partial(
    pl.kernel,
    out_type=jax.ShapeDtypeStruct((_B,), jnp.float32),
    mesh=plsc.VectorSubcoreMesh(core_axis_name="c", subcore_axis_name="s"),
    compiler_params=pltpu.CompilerParams(
        needs_layout_passes=False, use_tc_tiling_on_sc=True),
    scratch_types=[
        pltpu.VMEM((_BPW,), jnp.int32),
        pltpu.VMEM((_BPW,), jnp.int32),
        pltpu.VMEM((4, 8, _RPP * _L), jnp.float32),
        pltpu.VMEM((_D, _BPW), jnp.float32),
        pltpu.VMEM((_BPW,), jnp.float32),
        pltpu.SemaphoreType.DMA,
        pltpu.SemaphoreType.DMA,
    ],
)(_body)


def kernel(lbl_indx, hash_map, crx_vect, preset):
    lbl = lbl_indx.astype(jnp.int32)
    hm = hash_map.astype(jnp.int32)
    crxt = crx_vect.T
    pt = preset.T.reshape(4, 8, _V)
    return _clf(lbl, hm, crxt, pt)


# speedup vs baseline: 4.2373x; 2.8128x over previous
import functools

import jax
import jax.numpy as jnp
from jax import lax
from jax.experimental import pallas as pl
from jax.experimental.pallas import tpu as pltpu
from jax.experimental.pallas import tpu_sc as plsc

_NC = 2
_NS = 16
_NW = _NC * _NS
_L = 16

_B = 16384
_D = 32
_V = 1000000
_BPW = _B // _NW
_CH = 128
_NCH = _BPW // _CH
_PH = 4
_RPP = _BPW // _PH
_NBLK = _RPP // _L


def _rsqrt16(x):
    i = plsc.bitcast(x, jnp.int32)
    y = plsc.bitcast(jnp.int32(0x5F3759DF) - (i >> 1), jnp.float32)
    for _ in range(3):
        y = y * (1.5 - 0.5 * x * y * y)
    return y


def _body(lbl_hbm, hash_hbm, crxt_hbm, pt_hbm, out_hbm,
          hidx, pidx, bkt, g, crxv, outv, sem, gsem):
    c = lax.axis_index("c")
    s = lax.axis_index("s")
    wid = s * _NC + c
    base = wid * _BPW

    pltpu.sync_copy(hash_hbm.at[pl.ds(base, _BPW)], hidx)

    cps = [pltpu.async_copy(lbl_hbm.at[hidx.at[pl.ds(k * _CH, _CH)]],
                            pidx.at[pl.ds(k * _CH, _CH)], sem)
           for k in range(_NCH)]
    for cp in cps:
        cp.wait()

    ccp = pltpu.async_copy(crxt_hbm.at[:, pl.ds(base, _BPW)], crxv, sem)

    iota = lax.iota(jnp.int32, _L)
    ccp.wait()

    def phase(ph, pcarry):
        pbase = ph * _RPP

        def build(b, cnts):
            vec = pidx[pl.ds(pbase + b * _L, _L)]
            p = (vec >> 7) * 256 + (b * _L + iota)
            sgr = (vec >> 4) & 7
            new = []
            for sb in range(8):
                m = sgr == sb
                plsc.store_compressed(bkt.at[sb, pl.ds(cnts[sb], _L)],
                                      p, mask=m)
                pc = plsc.all_reduce_population_count(m)
                new.append(cnts[sb] + pc[0])
            return tuple(new)

        zero = jnp.int32(0)
        cnts = lax.fori_loop(0, _NBLK, build, (zero,) * 8)

        nchunks = []
        for sb in range(8):
            n = cnts[sb]
            bkt[sb, pl.ds(n, _L)] = jnp.full((_L,), _RPP, jnp.int32)
            nchunk = (n + _L - 1) >> 4
            nchunks.append(nchunk)

            def chunk(k, carry, sb=sb):
                ch = bkt[sb, pl.ds(k * _L, _L)]
                for t in range(_L):
                    pv = ch[t]
                    al = pl.multiple_of((pv >> 8) * 128, 128)
                    r = pv & 255
                    win = pt_hbm.at[:, :, pl.ds(al, 128)]
                    pltpu.async_copy(win.at[:, :, pl.ds(sb * _L, _L)],
                                     g.at[:, :, pl.ds(r * _L, _L)], gsem)
                return carry

            lax.fori_loop(0, nchunk, chunk, 0)

            if sb > 0:
                def drain(k, carry):
                    pltpu.make_async_copy(
                        pt_hbm.at[:, :, pl.ds(0, _L * _L)],
                        g.at[:, :, pl.ds(0, _L * _L)], gsem).wait()
                    return carry
                lax.fori_loop(0, nchunks[sb - 1], drain, 0)

        def drain(k, carry):
            pltpu.make_async_copy(
                pt_hbm.at[:, :, pl.ds(0, _L * _L)],
                g.at[:, :, pl.ds(0, _L * _L)], gsem).wait()
            return carry
        lax.fori_loop(0, nchunks[7], drain, 0)

        def blk(b, carry):
            o = pbase + b * _L
            low = pidx[pl.ds(o, _L)] & (_L - 1)
            lane = b * (_L * _L) + iota * _L + low
            acc_d = jnp.zeros((_L,), jnp.float32)
            acc_s = jnp.zeros((_L,), jnp.float32)
            for j in range(_D):
                a = crxv[j, pl.ds(o, _L)]
                p = plsc.load_gather(
                    g, [jnp.full((_L,), j // 8, jnp.int32),
                        jnp.full((_L,), j % 8, jnp.int32), lane])
                acc_s = acc_s + a * a
                acc_d = acc_d + a * p
            outv[pl.ds(o, _L)] = acc_d * _rsqrt16(acc_s)
            return carry

        lax.fori_loop(0, _NBLK, blk, 0)
        return pcarry

    lax.fori_loop(0, _PH, phase, 0)

    pltpu.sync_copy(outv, out_hbm.at[pl.ds(base, _BPW)])


_clf = functools.partial(
    pl.kernel,
    out_type=jax.ShapeDtypeStruct((_B,), jnp.float32),
    mesh=plsc.VectorSubcoreMesh(core_axis_name="c", subcore_axis_name="s"),
    compiler_params=pltpu.CompilerParams(
        needs_layout_passes=False, use_tc_tiling_on_sc=True),
    scratch_types=[
        pltpu.VMEM((_BPW,), jnp.int32),
        pltpu.VMEM((_BPW,), jnp.int32),
        pltpu.VMEM((8, _RPP + _L), jnp.int32),
        pltpu.VMEM((4, 8, (_RPP + 1) * _L), jnp.float32),
        pltpu.VMEM((_D, _BPW), jnp.float32),
        pltpu.VMEM((_BPW,), jnp.float32),
        pltpu.SemaphoreType.DMA,
        pltpu.SemaphoreType.DMA,
    ],
)(_body)


def kernel(lbl_indx, hash_map, crx_vect, preset):
    lbl = lbl_indx.astype(jnp.int32)
    hm = hash_map.astype(jnp.int32)
    crxt = crx_vect.T
    pt = preset.T.reshape(4, 8, _V)
    return _clf(lbl, hm, crxt, pt)


# speedup vs baseline: 4.2729x; 1.0084x over previous
import functools

import jax
import jax.numpy as jnp
from jax import lax
from jax.experimental import pallas as pl
from jax.experimental.pallas import tpu as pltpu
from jax.experimental.pallas import tpu_sc as plsc

_NC = 2
_NS = 16
_NW = _NC * _NS
_L = 16

_B = 16384
_D = 32
_V = 1000000
_BPW = _B // _NW
_CH = 128
_NCH = _BPW // _CH
_PH = 4
_RPP = _BPW // _PH
_NBLK = _RPP // _L


def _rsqrt16(x):
    i = plsc.bitcast(x, jnp.int32)
    y = plsc.bitcast(jnp.int32(0x5F3759DF) - (i >> 1), jnp.float32)
    for _ in range(3):
        y = y * (1.5 - 0.5 * x * y * y)
    return y


def _body(lbl_hbm, hash_hbm, crxt_hbm, pt_hbm, out_hbm,
          hidx, pidx, bkt, g, crxv, outv, sem, gsem):
    c = lax.axis_index("c")
    s = lax.axis_index("s")
    wid = s * _NC + c
    base = wid * _BPW

    pltpu.sync_copy(hash_hbm.at[pl.ds(base, _BPW)], hidx)

    cps = [pltpu.async_copy(lbl_hbm.at[hidx.at[pl.ds(k * _CH, _CH)]],
                            pidx.at[pl.ds(k * _CH, _CH)], sem)
           for k in range(_NCH)]
    for cp in cps:
        cp.wait()

    ccp = pltpu.async_copy(crxt_hbm.at[:, pl.ds(base, _BPW)], crxv, sem)

    iota = lax.iota(jnp.int32, _L)
    ccp.wait()

    def phase(ph, pcarry):
        pbase = ph * _RPP

        def build(b, cnts):
            vec = pidx[pl.ds(pbase + b * _L, _L)]
            p = (vec >> 7) * 256 + (b * _L + iota)
            sgr = (vec >> 4) & 7
            new = []
            for sb in range(8):
                m = sgr == sb
                plsc.store_compressed(bkt.at[sb, pl.ds(cnts[sb], _L)],
                                      p, mask=m)
                pc = plsc.all_reduce_population_count(m)
                new.append(cnts[sb] + pc[0])
            return tuple(new)

        zero = jnp.int32(0)
        cnts = lax.fori_loop(0, _NBLK, build, (zero,) * 8)

        nchunks = []
        for sb in range(8):
            n = cnts[sb]
            bkt[sb, pl.ds(n, _L)] = jnp.full((_L,), _RPP, jnp.int32)
            nchunk = (n + _L - 1) >> 4
            nchunks.append(nchunk)

            def chunk(k, carry, sb=sb):
                ch = bkt[sb, pl.ds(k * _L, _L)]
                for t in range(_L):
                    pv = ch[t]
                    al = pl.multiple_of((pv >> 8) * 128, 128)
                    r = pv & 255
                    win = pt_hbm.at[:, :, pl.ds(al, 128)]
                    pltpu.async_copy(win.at[:, :, pl.ds(sb * _L, _L)],
                                     g.at[:, :, pl.ds(r * _L, _L)], gsem)
                return carry

            lax.fori_loop(0, nchunk, chunk, 0)

        def drain(k, carry):
            pltpu.make_async_copy(
                pt_hbm.at[:, :, pl.ds(0, _L * _L)],
                g.at[:, :, pl.ds(0, _L * _L)], gsem).wait()
            return carry
        lax.fori_loop(0, sum(nchunks), drain, 0)

        def blk(b, carry):
            o = pbase + b * _L
            low = pidx[pl.ds(o, _L)] & (_L - 1)
            lane = b * (_L * _L) + iota * _L + low
            acc_d = jnp.zeros((_L,), jnp.float32)
            acc_s = jnp.zeros((_L,), jnp.float32)
            for j in range(_D):
                a = crxv[j, pl.ds(o, _L)]
                p = plsc.load_gather(
                    g, [jnp.full((_L,), j // 8, jnp.int32),
                        jnp.full((_L,), j % 8, jnp.int32), lane])
                acc_s = acc_s + a * a
                acc_d = acc_d + a * p
            outv[pl.ds(o, _L)] = acc_d * _rsqrt16(acc_s)
            return carry

        lax.fori_loop(0, _NBLK, blk, 0)
        return pcarry

    lax.fori_loop(0, _PH, phase, 0)

    pltpu.sync_copy(outv, out_hbm.at[pl.ds(base, _BPW)])


_clf = functools.partial(
    pl.kernel,
    out_type=jax.ShapeDtypeStruct((_B,), jnp.float32),
    mesh=plsc.VectorSubcoreMesh(core_axis_name="c", subcore_axis_name="s"),
    compiler_params=pltpu.CompilerParams(
        needs_layout_passes=False, use_tc_tiling_on_sc=True),
    scratch_types=[
        pltpu.VMEM((_BPW,), jnp.int32),
        pltpu.VMEM((_BPW,), jnp.int32),
        pltpu.VMEM((8, _RPP + _L), jnp.int32),
        pltpu.VMEM((4, 8, (_RPP + 1) * _L), jnp.float32),
        pltpu.VMEM((_D, _BPW), jnp.float32),
        pltpu.VMEM((_BPW,), jnp.float32),
        pltpu.SemaphoreType.DMA,
        pltpu.SemaphoreType.DMA,
    ],
)(_body)


def kernel(lbl_indx, hash_map, crx_vect, preset):
    lbl = lbl_indx.astype(jnp.int32)
    hm = hash_map.astype(jnp.int32)
    crxt = crx_vect.T
    pt = preset.T.reshape(4, 8, _V)
    return _clf(lbl, hm, crxt, pt)


# speedup vs baseline: 6.3780x; 1.4927x over previous
import functools

import jax
import jax.numpy as jnp
from jax import lax
from jax.experimental import pallas as pl
from jax.experimental.pallas import tpu as pltpu
from jax.experimental.pallas import tpu_sc as plsc

_NC = 2
_NS = 16
_NW = _NC * _NS
_L = 16

_B = 16384
_D = 32
_V = 1000000
_BPW = _B // _NW
_CH = 128
_NCH = _BPW // _CH
_PH = 4
_RPP = _BPW // _PH
_NBLK = _RPP // _L


def _rsqrt16(x):
    i = plsc.bitcast(x, jnp.int32)
    y = plsc.bitcast(jnp.int32(0x5F3759DF) - (i >> 1), jnp.float32)
    for _ in range(3):
        y = y * (1.5 - 0.5 * x * y * y)
    return y


def _body(lbl_hbm, hash_hbm, crxt_hbm, pt_hbm, out_hbm,
          hidx, pidx, bkt, g, crxv, outv, sem, gsem):
    c = lax.axis_index("c")
    s = lax.axis_index("s")
    wid = s * _NC + c
    base = wid * _BPW

    pltpu.sync_copy(hash_hbm.at[pl.ds(base, _BPW)], hidx)

    cps = [pltpu.async_copy(lbl_hbm.at[hidx.at[pl.ds(k * _CH, _CH)]],
                            pidx.at[pl.ds(k * _CH, _CH)], sem)
           for k in range(_NCH)]
    for cp in cps:
        cp.wait()

    ccp = pltpu.async_copy(crxt_hbm.at[:, pl.ds(base, _BPW)], crxv, sem)

    iota = lax.iota(jnp.int32, _L)
    ccp.wait()

    def phase(ph, pcarry):
        pbase = ph * _RPP

        def build(b, cnts):
            vec = pidx[pl.ds(pbase + b * _L, _L)]
            p = (vec >> 7) * 256 + (b * _L + iota)
            sgr = (vec >> 4) & 7
            new = []
            for sb in range(8):
                m = sgr == sb
                plsc.store_compressed(bkt.at[sb, pl.ds(cnts[sb], _L)],
                                      p, mask=m)
                pc = plsc.all_reduce_population_count(m)
                new.append(cnts[sb] + pc[0])
            return tuple(new)

        zero = jnp.int32(0)
        cnts = lax.fori_loop(0, _NBLK, build, (zero,) * 8)

        nchunks = []
        for sb in range(8):
            n = cnts[sb]
            bkt[sb, pl.ds(n, _L)] = jnp.full((_L,), _RPP, jnp.int32)
            nchunk = (n + 7) >> 3
            nchunks.append(nchunk)

            def chunk(k, carry, sb=sb):
                ch = bkt[sb, pl.ds(k * 8, _L)]
                for t in range(8):
                    pv = ch[t]
                    al = pl.multiple_of((pv >> 8) * 128, 128)
                    r = pv & 255
                    win = pt_hbm.at[:, :, pl.ds(al, 128)]
                    pltpu.async_copy(win.at[:, :, pl.ds(sb * _L, _L)],
                                     g.at[:, :, pl.ds(r * _L, _L)], gsem)
                return carry

            lax.fori_loop(0, nchunk, chunk, 0)

        def drain(k, carry):
            pltpu.make_async_copy(
                pt_hbm.at[:, :, pl.ds(0, 8 * _L)],
                g.at[:, :, pl.ds(0, 8 * _L)], gsem).wait()
            return carry
        lax.fori_loop(0, sum(nchunks), drain, 0)

        def blk(b, carry):
            o = pbase + b * _L
            low = pidx[pl.ds(o, _L)] & (_L - 1)
            lane = b * (_L * _L) + iota * _L + low
            acc_d = jnp.zeros((_L,), jnp.float32)
            acc_s = jnp.zeros((_L,), jnp.float32)
            for j in range(_D):
                a = crxv[j, pl.ds(o, _L)]
                p = plsc.load_gather(
                    g, [jnp.full((_L,), j // 8, jnp.int32),
                        jnp.full((_L,), j % 8, jnp.int32), lane])
                acc_s = acc_s + a * a
                acc_d = acc_d + a * p
            outv[pl.ds(o, _L)] = acc_d * _rsqrt16(acc_s)
            return carry

        lax.fori_loop(0, _NBLK, blk, 0)
        return pcarry

    lax.fori_loop(0, _PH, phase, 0)

    pltpu.sync_copy(outv, out_hbm.at[pl.ds(base, _BPW)])


_clf = functools.partial(
    pl.kernel,
    out_type=jax.ShapeDtypeStruct((_B,), jnp.float32),
    mesh=plsc.VectorSubcoreMesh(core_axis_name="c", subcore_axis_name="s"),
    compiler_params=pltpu.CompilerParams(
        needs_layout_passes=False, use_tc_tiling_on_sc=True),
    scratch_types=[
        pltpu.VMEM((_BPW,), jnp.int32),
        pltpu.VMEM((_BPW,), jnp.int32),
        pltpu.VMEM((8, _RPP + _L), jnp.int32),
        pltpu.VMEM((4, 8, (_RPP + 1) * _L), jnp.float32),
        pltpu.VMEM((_D, _BPW), jnp.float32),
        pltpu.VMEM((_BPW,), jnp.float32),
        pltpu.SemaphoreType.DMA,
        pltpu.SemaphoreType.DMA,
    ],
)(_body)


def kernel(lbl_indx, hash_map, crx_vect, preset):
    lbl = lbl_indx.astype(jnp.int32)
    hm = hash_map.astype(jnp.int32)
    crxt = crx_vect.T
    pt = preset.T.reshape(4, 8, _V)
    return _clf(lbl, hm, crxt, pt)


# speedup vs baseline: 8.7794x; 1.3765x over previous
import functools

import jax
import jax.numpy as jnp
from jax import lax
from jax.experimental import pallas as pl
from jax.experimental.pallas import tpu as pltpu
from jax.experimental.pallas import tpu_sc as plsc

_NC = 2
_NS = 16
_NW = _NC * _NS
_L = 16

_B = 16384
_D = 32
_V = 1000000
_BPW = _B // _NW
_CH = 128
_NCH = _BPW // _CH
_PH = 4
_RPP = _BPW // _PH
_NBLK = _RPP // _L


def _rsqrt16(x):
    i = plsc.bitcast(x, jnp.int32)
    y = plsc.bitcast(jnp.int32(0x5F3759DF) - (i >> 1), jnp.float32)
    for _ in range(3):
        y = y * (1.5 - 0.5 * x * y * y)
    return y


def _body(lbl_hbm, hash_hbm, crxt_hbm, pt_hbm, out_hbm,
          hidx, pidx, bkt, g, crxv, outv, sem, gsem):
    c = lax.axis_index("c")
    s = lax.axis_index("s")
    wid = s * _NC + c
    base = wid * _BPW

    pltpu.sync_copy(hash_hbm.at[pl.ds(base, _BPW)], hidx)

    cps = [pltpu.async_copy(lbl_hbm.at[hidx.at[pl.ds(k * _CH, _CH)]],
                            pidx.at[pl.ds(k * _CH, _CH)], sem)
           for k in range(_NCH)]
    for cp in cps:
        cp.wait()

    ccp = pltpu.async_copy(crxt_hbm.at[:, pl.ds(base, _BPW)], crxv, sem)

    iota = lax.iota(jnp.int32, _L)
    ccp.wait()

    def phase(ph, pcarry):
        pbase = ph * _RPP

        def build(b, cnts):
            vec = pidx[pl.ds(pbase + b * _L, _L)]
            p = (vec >> 7) * 256 + (b * _L + iota)
            sgr = (vec >> 4) & 7
            new = []
            for sb in range(8):
                m = sgr == sb
                plsc.store_compressed(bkt.at[sb, pl.ds(cnts[sb], _L)],
                                      p, mask=m)
                pc = plsc.all_reduce_population_count(m)
                new.append(cnts[sb] + pc[0])
            return tuple(new)

        zero = jnp.int32(0)
        cnts = lax.fori_loop(0, _NBLK, build, (zero,) * 8)

        nchunks = []
        for sb in range(8):
            n = cnts[sb]
            bkt[sb, pl.ds(n, _L)] = jnp.full((_L,), _RPP, jnp.int32)
            nchunk = (n + 3) >> 2
            nchunks.append(nchunk)

            def chunk(k, carry, sb=sb):
                ch = bkt[sb, pl.ds(k * 4, _L)]
                for t in range(4):
                    pv = ch[t]
                    al = pl.multiple_of((pv >> 8) * 128, 128)
                    r = pv & 255
                    win = pt_hbm.at[:, :, pl.ds(al, 128)]
                    pltpu.async_copy(win.at[:, :, pl.ds(sb * _L, _L)],
                                     g.at[:, :, pl.ds(r * _L, _L)], gsem)
                return carry

            lax.fori_loop(0, nchunk, chunk, 0)

        def drain(k, carry):
            pltpu.make_async_copy(
                pt_hbm.at[:, :, pl.ds(0, 4 * _L)],
                g.at[:, :, pl.ds(0, 4 * _L)], gsem).wait()
            return carry
        lax.fori_loop(0, sum(nchunks), drain, 0)

        def blk(b, carry):
            o = pbase + b * _L
            low = pidx[pl.ds(o, _L)] & (_L - 1)
            lane = b * (_L * _L) + iota * _L + low
            acc_d = jnp.zeros((_L,), jnp.float32)
            acc_s = jnp.zeros((_L,), jnp.float32)
            for j in range(_D):
                a = crxv[j, pl.ds(o, _L)]
                p = plsc.load_gather(
                    g, [jnp.full((_L,), j // 8, jnp.int32),
                        jnp.full((_L,), j % 8, jnp.int32), lane])
                acc_s = acc_s + a * a
                acc_d = acc_d + a * p
            outv[pl.ds(o, _L)] = acc_d * _rsqrt16(acc_s)
            return carry

        lax.fori_loop(0, _NBLK, blk, 0)
        return pcarry

    lax.fori_loop(0, _PH, phase, 0)

    pltpu.sync_copy(outv, out_hbm.at[pl.ds(base, _BPW)])


_clf = functools.partial(
    pl.kernel,
    out_type=jax.ShapeDtypeStruct((_B,), jnp.float32),
    mesh=plsc.VectorSubcoreMesh(core_axis_name="c", subcore_axis_name="s"),
    compiler_params=pltpu.CompilerParams(
        needs_layout_passes=False, use_tc_tiling_on_sc=True),
    scratch_types=[
        pltpu.VMEM((_BPW,), jnp.int32),
        pltpu.VMEM((_BPW,), jnp.int32),
        pltpu.VMEM((8, _RPP + _L), jnp.int32),
        pltpu.VMEM((4, 8, (_RPP + 1) * _L), jnp.float32),
        pltpu.VMEM((_D, _BPW), jnp.float32),
        pltpu.VMEM((_BPW,), jnp.float32),
        pltpu.SemaphoreType.DMA,
        pltpu.SemaphoreType.DMA,
    ],
)(_body)


def kernel(lbl_indx, hash_map, crx_vect, preset):
    lbl = lbl_indx.astype(jnp.int32)
    hm = hash_map.astype(jnp.int32)
    crxt = crx_vect.T
    pt = preset.T.reshape(4, 8, _V)
    return _clf(lbl, hm, crxt, pt)


# speedup vs baseline: 9.4749x; 1.0792x over previous
import functools

import jax
import jax.numpy as jnp
from jax import lax
from jax.experimental import pallas as pl
from jax.experimental.pallas import tpu as pltpu
from jax.experimental.pallas import tpu_sc as plsc

_NC = 2
_NS = 16
_NW = _NC * _NS
_L = 16

_B = 16384
_D = 32
_V = 1000000
_BPW = _B // _NW
_CH = 128
_NCH = _BPW // _CH
_PH = 4
_RPP = _BPW // _PH
_NBLK = _RPP // _L


def _rsqrt16(x):
    i = plsc.bitcast(x, jnp.int32)
    y = plsc.bitcast(jnp.int32(0x5F3759DF) - (i >> 1), jnp.float32)
    for _ in range(3):
        y = y * (1.5 - 0.5 * x * y * y)
    return y


def _body(lbl_hbm, hash_hbm, crxt_hbm, pt_hbm, out_hbm,
          hidx, pidx, bkt, g, crxv, outv, sem, gsem):
    c = lax.axis_index("c")
    s = lax.axis_index("s")
    wid = s * _NC + c
    base = wid * _BPW

    pltpu.sync_copy(hash_hbm.at[pl.ds(base, _BPW)], hidx)

    cps = [pltpu.async_copy(lbl_hbm.at[hidx.at[pl.ds(k * _CH, _CH)]],
                            pidx.at[pl.ds(k * _CH, _CH)], sem)
           for k in range(_NCH)]
    for cp in cps:
        cp.wait()

    ccp = pltpu.async_copy(crxt_hbm.at[:, pl.ds(base, _BPW)], crxv, sem)

    iota = lax.iota(jnp.int32, _L)
    ccp.wait()

    def phase(ph, pcarry):
        pbase = ph * _RPP

        def build(b, cnts):
            vec = pidx[pl.ds(pbase + b * _L, _L)]
            p = (vec >> 7) * 256 + (b * _L + iota)
            sgr = (vec >> 4) & 7
            new = []
            for sb in range(8):
                m = sgr == sb
                plsc.store_compressed(bkt.at[sb, pl.ds(cnts[sb], _L)],
                                      p, mask=m)
                pc = plsc.all_reduce_population_count(m)
                new.append(cnts[sb] + pc[0])
            return tuple(new)

        zero = jnp.int32(0)
        cnts = lax.fori_loop(0, _NBLK, build, (zero,) * 8)

        nchunks = []
        for sb in range(8):
            n = cnts[sb]
            bkt[sb, pl.ds(n, _L)] = jnp.full((_L,), _RPP, jnp.int32)
            nchunk = (n + 1) >> 1
            nchunks.append(nchunk)

            def chunk(k, carry, sb=sb):
                ch = bkt[sb, pl.ds(k * 2, _L)]
                for t in range(2):
                    pv = ch[t]
                    al = pl.multiple_of((pv >> 8) * 128, 128)
                    r = pv & 255
                    win = pt_hbm.at[:, :, pl.ds(al, 128)]
                    pltpu.async_copy(win.at[:, :, pl.ds(sb * _L, _L)],
                                     g.at[:, :, pl.ds(r * _L, _L)], gsem)
                return carry

            lax.fori_loop(0, nchunk, chunk, 0)

        def drain(k, carry):
            pltpu.make_async_copy(
                pt_hbm.at[:, :, pl.ds(0, 2 * _L)],
                g.at[:, :, pl.ds(0, 2 * _L)], gsem).wait()
            return carry
        lax.fori_loop(0, sum(nchunks), drain, 0)

        def blk(b, carry):
            o = pbase + b * _L
            low = pidx[pl.ds(o, _L)] & (_L - 1)
            lane = b * (_L * _L) + iota * _L + low
            acc_d = jnp.zeros((_L,), jnp.float32)
            acc_s = jnp.zeros((_L,), jnp.float32)
            for j in range(_D):
                a = crxv[j, pl.ds(o, _L)]
                p = plsc.load_gather(
                    g, [jnp.full((_L,), j // 8, jnp.int32),
                        jnp.full((_L,), j % 8, jnp.int32), lane])
                acc_s = acc_s + a * a
                acc_d = acc_d + a * p
            outv[pl.ds(o, _L)] = acc_d * _rsqrt16(acc_s)
            return carry

        lax.fori_loop(0, _NBLK, blk, 0)
        return pcarry

    lax.fori_loop(0, _PH, phase, 0)

    pltpu.sync_copy(outv, out_hbm.at[pl.ds(base, _BPW)])


_clf = functools.partial(
    pl.kernel,
    out_type=jax.ShapeDtypeStruct((_B,), jnp.float32),
    mesh=plsc.VectorSubcoreMesh(core_axis_name="c", subcore_axis_name="s"),
    compiler_params=pltpu.CompilerParams(
        needs_layout_passes=False, use_tc_tiling_on_sc=True),
    scratch_types=[
        pltpu.VMEM((_BPW,), jnp.int32),
        pltpu.VMEM((_BPW,), jnp.int32),
        pltpu.VMEM((8, _RPP + _L), jnp.int32),
        pltpu.VMEM((4, 8, (_RPP + 1) * _L), jnp.float32),
        pltpu.VMEM((_D, _BPW), jnp.float32),
        pltpu.VMEM((_BPW,), jnp.float32),
        pltpu.SemaphoreType.DMA,
        pltpu.SemaphoreType.DMA,
    ],
)(_body)


def kernel(lbl_indx, hash_map, crx_vect, preset):
    lbl = lbl_indx.astype(jnp.int32)
    hm = hash_map.astype(jnp.int32)
    crxt = crx_vect.T
    pt = preset.T.reshape(4, 8, _V)
    return _clf(lbl, hm, crxt, pt)
